# Initial kernel scaffold; baseline (speedup 1.0000x reference)
import jax
import jax.numpy as jnp
from jax.experimental import pallas as pl


def kernel(token_ids, word_ids, attention_mask, emb1, emb2, W1, b1, gamma, beta, Wf, bf):
    raise NotImplementedError("write your pallas kernel here")


# speedup vs baseline: 3.9815x; 3.9815x over previous
import functools

import jax
import jax.numpy as jnp
from jax import lax
from jax.experimental import pallas as pl
from jax.experimental.pallas import tpu as pltpu
from jax.experimental.pallas import tpu_sc as plsc

_B, _L = 4096, 50
_N = _B * _L
_D = 64
_H = 256
_C = 1000
_EPS = 1e-5

_NW = 32
_PER_W = _N // _NW
_CH = 128
_NCH = _PER_W // _CH

_RB = 4096
_NSTEP = _N // _RB

_BB = 128
_NB = _B // _BB


def _sc_gather(tok, word, emb1, emb2):
    mesh = plsc.VectorSubcoreMesh(core_axis_name="c", subcore_axis_name="s")

    @functools.partial(
        pl.kernel, mesh=mesh,
        compiler_params=pltpu.CompilerParams(use_tc_tiling_on_sc=False),
        out_type=[
            jax.ShapeDtypeStruct((_N, _D), jnp.float32),
            jax.ShapeDtypeStruct((_N, _D), jnp.float32),
        ],
        scratch_types=[
            pltpu.VMEM((_NCH, _CH), jnp.int32),
            pltpu.VMEM((_NCH, _CH), jnp.int32),
            pltpu.VMEM((_CH, _D), jnp.float32),
            pltpu.VMEM((_CH, _D), jnp.float32),
            pltpu.SemaphoreType.DMA,
            pltpu.SemaphoreType.DMA,
        ],
    )
    def k(tok_hbm, word_hbm, e1_hbm, e2_hbm, out1_hbm, out2_hbm,
          idx1_v, idx2_v, buf1, buf2, sem1, sem2):
        wid = lax.axis_index("s") * 2 + lax.axis_index("c")
        base = wid * _PER_W
        pltpu.sync_copy(tok_hbm.at[wid], idx1_v)
        pltpu.sync_copy(word_hbm.at[wid], idx2_v)

        def body(j, carry):
            c1 = pltpu.async_copy(e1_hbm.at[idx1_v.at[j]], buf1, sem1)
            c2 = pltpu.async_copy(e2_hbm.at[idx2_v.at[j]], buf2, sem2)
            c1.wait()
            c2.wait()
            r0 = base + j * _CH
            pltpu.sync_copy(buf1, out1_hbm.at[pl.ds(r0, _CH)])
            pltpu.sync_copy(buf2, out2_hbm.at[pl.ds(r0, _CH)])
            return carry

        lax.fori_loop(0, _NCH, body, 0)

    return k(tok, word, emb1, emb2)


def _stats(flat1, flat2, w1t, gamma, beta):
    def kern(a_ref, b_ref, w_ref, g_ref, be_ref, out_ref, m_acc, s_acc):
        i = pl.program_id(0)

        @pl.when(i == 0)
        def _init():
            m_acc[...] = jnp.zeros_like(m_acc)
            s_acc[...] = jnp.zeros_like(s_acc)

        x = jnp.concatenate([a_ref[...], b_ref[...]], axis=-1)
        m_acc[...] += lax.dot_general(
            x, x, (((0,), (0,)), ((), ())), preferred_element_type=jnp.float32)
        s_acc[...] += jnp.sum(x, axis=0, keepdims=True)

        @pl.when(i == _NSTEP - 1)
        def _fin():
            w = w_ref[...]
            m = jnp.dot(s_acc[...], w,
                        preferred_element_type=jnp.float32) / _N
            u = jnp.dot(m_acc[...], w,
                        preferred_element_type=jnp.float32)
            t = jnp.sum(w * u, axis=0, keepdims=True) / _N
            var = t - m * m
            scale = g_ref[...] * lax.rsqrt(var + _EPS)
            shift = be_ref[...] - m * scale
            out_ref[0:1, :] = scale
            out_ref[1:2, :] = shift

    return pl.pallas_call(
        kern,
        grid=(_NSTEP,),
        in_specs=[
            pl.BlockSpec((_RB, _D), lambda i: (i, 0)),
            pl.BlockSpec((_RB, _D), lambda i: (i, 0)),
            pl.BlockSpec((2 * _D, _H), lambda i: (0, 0)),
            pl.BlockSpec((1, _H), lambda i: (0, 0)),
            pl.BlockSpec((1, _H), lambda i: (0, 0)),
        ],
        out_specs=pl.BlockSpec((2, _H), lambda i: (0, 0)),
        out_shape=jax.ShapeDtypeStruct((2, _H), jnp.float32),
        scratch_shapes=[
            pltpu.VMEM((2 * _D, 2 * _D), jnp.float32),
            pltpu.VMEM((1, 2 * _D), jnp.float32),
        ],
    )(flat1, flat2, w1t, gamma, beta)


def _main(flat1, flat2, mask, ss, w1t, wft, bfr):
    def kern(a_ref, b_ref, m_ref, ss_ref, w_ref, wf_ref, bf_ref, o_ref):
        a = a_ref[...].reshape(_BB * _L, _D)
        b = b_ref[...].reshape(_BB * _L, _D)
        x = jnp.concatenate([a, b], axis=-1)
        y = jnp.dot(x, w_ref[...], preferred_element_type=jnp.float32)
        h = jnp.maximum(y * ss_ref[0:1, :] + ss_ref[1:2, :], 0.0)
        pooled = jnp.sum(h.reshape(_BB, _L, _H), axis=1)
        denom = jnp.sum(m_ref[...], axis=1, keepdims=True)
        pooled = pooled / denom
        o_ref[...] = jnp.dot(pooled, wf_ref[...],
                             preferred_element_type=jnp.float32) + bf_ref[...]

    return pl.pallas_call(
        kern,
        grid=(_NB,),
        in_specs=[
            pl.BlockSpec((_BB, _L, _D), lambda i: (i, 0, 0)),
            pl.BlockSpec((_BB, _L, _D), lambda i: (i, 0, 0)),
            pl.BlockSpec((_BB, _L), lambda i: (i, 0)),
            pl.BlockSpec((2, _H), lambda i: (0, 0)),
            pl.BlockSpec((2 * _D, _H), lambda i: (0, 0)),
            pl.BlockSpec((_H, _C), lambda i: (0, 0)),
            pl.BlockSpec((1, _C), lambda i: (0, 0)),
        ],
        out_specs=pl.BlockSpec((_BB, _C), lambda i: (i, 0)),
        out_shape=jax.ShapeDtypeStruct((_B, _C), jnp.float32),
    )(flat1, flat2, mask, ss, w1t, wft, bfr)


def kernel(token_ids, word_ids, attention_mask, emb1, emb2, W1, b1, gamma,
           beta, Wf, bf):
    del b1
    tok = token_ids.reshape(_NW, _NCH, _CH).astype(jnp.int32)
    word = word_ids.reshape(_NW, _NCH, _CH).astype(jnp.int32)
    flat1, flat2 = _sc_gather(tok, word, emb1, emb2)
    w1t = W1.T
    ss = _stats(flat1, flat2, w1t, gamma.reshape(1, _H), beta.reshape(1, _H))
    logits = _main(flat1.reshape(_B, _L, _D), flat2.reshape(_B, _L, _D),
                   attention_mask, ss, w1t, Wf.T, bf.reshape(1, _C))
    return logits


# speedup vs baseline: 6.9469x; 1.7448x over previous
import functools

import jax
import jax.numpy as jnp
from jax import lax
from jax.experimental import pallas as pl
from jax.experimental.pallas import tpu as pltpu
from jax.experimental.pallas import tpu_sc as plsc

_B, _L = 4096, 50
_N = _B * _L
_D = 64
_H = 256
_C = 1000
_EPS = 1e-5

_NW = 32
_PER_W = _N // _NW
_CH = 128
_NCH = _PER_W // _CH

_RB = 4096
_NSTEP = _N // _RB

_BB = 1024
_NB = _B // _BB


def _sc_gather(tok, word, emb1, emb2):
    mesh = plsc.VectorSubcoreMesh(core_axis_name="c", subcore_axis_name="s")

    @functools.partial(
        pl.kernel, mesh=mesh,
        compiler_params=pltpu.CompilerParams(use_tc_tiling_on_sc=False),
        out_type=jax.ShapeDtypeStruct((_N, 2 * _D), jnp.float32),
        scratch_types=[
            pltpu.VMEM((_NCH, _CH), jnp.int32),
            pltpu.VMEM((_NCH, _CH), jnp.int32),
            pltpu.VMEM((_CH, _D), jnp.float32),
            pltpu.VMEM((_CH, _D), jnp.float32),
            pltpu.SemaphoreType.DMA,
            pltpu.SemaphoreType.DMA,
        ],
    )
    def k(tok_hbm, word_hbm, e1_hbm, e2_hbm, out_hbm,
          idx1_v, idx2_v, buf1, buf2, sem1, sem2):
        wid = lax.axis_index("s") * 2 + lax.axis_index("c")
        base = wid * _PER_W
        pltpu.sync_copy(tok_hbm.at[wid], idx1_v)
        pltpu.sync_copy(word_hbm.at[wid], idx2_v)

        def body(j, carry):
            c1 = pltpu.async_copy(e1_hbm.at[idx1_v.at[j]], buf1, sem1)
            c2 = pltpu.async_copy(e2_hbm.at[idx2_v.at[j]], buf2, sem2)
            c1.wait()
            c2.wait()
            r0 = base + j * _CH
            pltpu.sync_copy(buf1, out_hbm.at[pl.ds(r0, _CH), pl.ds(0, _D)])
            pltpu.sync_copy(buf2, out_hbm.at[pl.ds(r0, _CH), pl.ds(_D, _D)])
            return carry

        lax.fori_loop(0, _NCH, body, 0)

    return k(tok, word, emb1, emb2)


def _stats(flat, w1t, gamma, beta):
    def kern(x_ref, w_ref, g_ref, be_ref, out_ref, m_acc, s_acc):
        i = pl.program_id(0)

        @pl.when(i == 0)
        def _init():
            m_acc[...] = jnp.zeros_like(m_acc)
            s_acc[...] = jnp.zeros_like(s_acc)

        x = x_ref[...]
        m_acc[...] += lax.dot_general(
            x, x, (((0,), (0,)), ((), ())), preferred_element_type=jnp.float32)
        s_acc[...] += jnp.sum(x, axis=0, keepdims=True)

        @pl.when(i == _NSTEP - 1)
        def _fin():
            w = w_ref[...]
            m = jnp.dot(s_acc[...], w,
                        preferred_element_type=jnp.float32) / _N
            u = jnp.dot(m_acc[...], w,
                        preferred_element_type=jnp.float32)
            t = jnp.sum(w * u, axis=0, keepdims=True) / _N
            var = t - m * m
            scale = g_ref[...] * lax.rsqrt(var + _EPS)
            shift = be_ref[...] - m * scale
            out_ref[0:1, :] = scale
            out_ref[1:2, :] = shift

    return pl.pallas_call(
        kern,
        grid=(_NSTEP,),
        in_specs=[
            pl.BlockSpec((_RB, 2 * _D), lambda i: (i, 0)),
            pl.BlockSpec((2 * _D, _H), lambda i: (0, 0)),
            pl.BlockSpec((1, _H), lambda i: (0, 0)),
            pl.BlockSpec((1, _H), lambda i: (0, 0)),
        ],
        out_specs=pl.BlockSpec((2, _H), lambda i: (0, 0)),
        out_shape=jax.ShapeDtypeStruct((2, _H), jnp.float32),
        scratch_shapes=[
            pltpu.VMEM((2 * _D, 2 * _D), jnp.float32),
            pltpu.VMEM((1, 2 * _D), jnp.float32),
        ],
    )(flat, w1t, gamma, beta)


def _main(flat, mask, ss, w1t, wft, bfr):
    def kern(x_ref, m_ref, ss_ref, w_ref, wf_ref, bf_ref, o_ref, acc):
        l = pl.program_id(1)
        ws = w_ref[...] * ss_ref[0:1, :]
        y = jnp.dot(x_ref[...], ws, preferred_element_type=jnp.float32)
        t = jnp.maximum(y + ss_ref[1:2, :], 0.0)

        @pl.when(l == 0)
        def _first():
            acc[...] = t

        @pl.when(l > 0)
        def _rest():
            acc[...] += t

        @pl.when(l == _L - 1)
        def _fin():
            denom = jnp.sum(m_ref[...], axis=1, keepdims=True)
            pooled = acc[...] / denom
            o_ref[...] = jnp.dot(pooled, wf_ref[...],
                                 preferred_element_type=jnp.float32) + bf_ref[...]

    return pl.pallas_call(
        kern,
        grid=(_NB, _L),
        in_specs=[
            pl.BlockSpec((_BB, 2 * _D), lambda i, l: (l * _NB + i, 0)),
            pl.BlockSpec((_BB, _L), lambda i, l: (i, 0)),
            pl.BlockSpec((2, _H), lambda i, l: (0, 0)),
            pl.BlockSpec((2 * _D, _H), lambda i, l: (0, 0)),
            pl.BlockSpec((_H, _C), lambda i, l: (0, 0)),
            pl.BlockSpec((1, _C), lambda i, l: (0, 0)),
        ],
        out_specs=pl.BlockSpec((_BB, _C), lambda i, l: (i, 0)),
        out_shape=jax.ShapeDtypeStruct((_B, _C), jnp.float32),
        scratch_shapes=[pltpu.VMEM((_BB, _H), jnp.float32)],
    )(flat, mask, ss, w1t, wft, bfr)


def kernel(token_ids, word_ids, attention_mask, emb1, emb2, W1, b1, gamma,
           beta, Wf, bf):
    del b1
    tok = token_ids.T.reshape(_NW, _NCH, _CH).astype(jnp.int32)
    word = word_ids.T.reshape(_NW, _NCH, _CH).astype(jnp.int32)
    flat = _sc_gather(tok, word, emb1, emb2)
    w1t = W1.T
    ss = _stats(flat, w1t, gamma.reshape(1, _H), beta.reshape(1, _H))
    logits = _main(flat, attention_mask, ss, w1t, Wf.T, bf.reshape(1, _C))
    return logits


# speedup vs baseline: 8.4406x; 1.2150x over previous
import functools

import jax
import jax.numpy as jnp
from jax import lax
from jax.experimental import pallas as pl
from jax.experimental.pallas import tpu as pltpu
from jax.experimental.pallas import tpu_sc as plsc

_B, _L = 4096, 50
_N = _B * _L
_D = 64
_H = 256
_C = 1000
_EPS = 1e-5

_NW = 32
_PER_W = _N // _NW
_CH = 128
_NCH = _PER_W // _CH

_RB = 4096
_NSTEP = _N // _RB

_BB = 2048
_NB = _B // _BB


def _sc_gather(tok, word, emb1, emb2):
    mesh = plsc.VectorSubcoreMesh(core_axis_name="c", subcore_axis_name="s")

    @functools.partial(
        pl.kernel, mesh=mesh,
        compiler_params=pltpu.CompilerParams(use_tc_tiling_on_sc=False),
        out_type=jax.ShapeDtypeStruct((_N, 2 * _D), jnp.float32),
        scratch_types=[
            pltpu.VMEM((_NCH, _CH), jnp.int32),
            pltpu.VMEM((_NCH, _CH), jnp.int32),
            pltpu.VMEM((2, _CH, _D), jnp.float32),
            pltpu.VMEM((2, _CH, _D), jnp.float32),
            pltpu.SemaphoreType.DMA,
            pltpu.SemaphoreType.DMA,
            pltpu.SemaphoreType.DMA,
            pltpu.SemaphoreType.DMA,
        ],
    )
    def k(tok_hbm, word_hbm, e1_hbm, e2_hbm, out_hbm,
          idx1_v, idx2_v, buf1, buf2, s1a, s1b, s2a, s2b):
        wid = lax.axis_index("s") * 2 + lax.axis_index("c")
        base = wid * _PER_W
        pltpu.sync_copy(tok_hbm.at[wid], idx1_v)
        pltpu.sync_copy(word_hbm.at[wid], idx2_v)

        def start(j, p, s1, s2):
            pltpu.async_copy(e1_hbm.at[idx1_v.at[j]], buf1.at[p], s1)
            pltpu.async_copy(e2_hbm.at[idx2_v.at[j]], buf2.at[p], s2)

        def wait_write(j, p, s1, s2):
            pltpu.make_async_copy(e1_hbm.at[idx1_v.at[j]], buf1.at[p], s1).wait()
            pltpu.make_async_copy(e2_hbm.at[idx2_v.at[j]], buf2.at[p], s2).wait()
            r0 = base + j * _CH
            pltpu.sync_copy(buf1.at[p], out_hbm.at[pl.ds(r0, _CH), pl.ds(0, _D)])
            pltpu.sync_copy(buf2.at[p], out_hbm.at[pl.ds(r0, _CH), pl.ds(_D, _D)])

        start(0, 0, s1a, s2a)

        def body(k2, carry):
            j0 = 2 * k2
            start(j0 + 1, 1, s1b, s2b)
            wait_write(j0, 0, s1a, s2a)

            @pl.when(j0 + 2 < _NCH)
            def _():
                start(j0 + 2, 0, s1a, s2a)

            wait_write(j0 + 1, 1, s1b, s2b)
            return carry

        lax.fori_loop(0, _NCH // 2, body, 0)

    return k(tok, word, emb1, emb2)


def _stats(flat, w1t, gamma, beta):
    def kern(x_ref, w_ref, g_ref, be_ref, out_ref, m_acc, s_acc):
        i = pl.program_id(0)

        @pl.when(i == 0)
        def _init():
            m_acc[...] = jnp.zeros_like(m_acc)
            s_acc[...] = jnp.zeros_like(s_acc)

        x = x_ref[...]
        m_acc[...] += lax.dot_general(
            x, x, (((0,), (0,)), ((), ())), preferred_element_type=jnp.float32)
        s_acc[...] += jnp.sum(x, axis=0, keepdims=True)

        @pl.when(i == _NSTEP - 1)
        def _fin():
            w = w_ref[...]
            m = jnp.dot(s_acc[...], w,
                        preferred_element_type=jnp.float32) / _N
            u = jnp.dot(m_acc[...], w,
                        preferred_element_type=jnp.float32)
            t = jnp.sum(w * u, axis=0, keepdims=True) / _N
            var = t - m * m
            scale = g_ref[...] * lax.rsqrt(var + _EPS)
            shift = be_ref[...] - m * scale
            out_ref[0:1, :] = scale
            out_ref[1:2, :] = shift

    return pl.pallas_call(
        kern,
        grid=(_NSTEP,),
        in_specs=[
            pl.BlockSpec((_RB, 2 * _D), lambda i: (i, 0)),
            pl.BlockSpec((2 * _D, _H), lambda i: (0, 0)),
            pl.BlockSpec((1, _H), lambda i: (0, 0)),
            pl.BlockSpec((1, _H), lambda i: (0, 0)),
        ],
        out_specs=pl.BlockSpec((2, _H), lambda i: (0, 0)),
        out_shape=jax.ShapeDtypeStruct((2, _H), jnp.float32),
        scratch_shapes=[
            pltpu.VMEM((2 * _D, 2 * _D), jnp.float32),
            pltpu.VMEM((1, 2 * _D), jnp.float32),
        ],
    )(flat, w1t, gamma, beta)


def _main(flat, mask, ss, w1t, wft, bfr):
    def kern(x_ref, m_ref, ss_ref, w_ref, wf_ref, bf_ref, o_ref, acc):
        l = pl.program_id(1)
        ws = w_ref[...] * ss_ref[0:1, :]
        y = jnp.dot(x_ref[...], ws, preferred_element_type=jnp.float32)
        t = jnp.maximum(y + ss_ref[1:2, :], 0.0)

        @pl.when(l == 0)
        def _first():
            acc[...] = t

        @pl.when(l > 0)
        def _rest():
            acc[...] += t

        @pl.when(l == _L - 1)
        def _fin():
            denom = jnp.sum(m_ref[...], axis=1, keepdims=True)
            pooled = acc[...] / denom
            o_ref[...] = jnp.dot(pooled, wf_ref[...],
                                 preferred_element_type=jnp.float32) + bf_ref[...]

    return pl.pallas_call(
        kern,
        grid=(_NB, _L),
        in_specs=[
            pl.BlockSpec((_BB, 2 * _D), lambda i, l: (l * _NB + i, 0)),
            pl.BlockSpec((_BB, _L), lambda i, l: (i, 0)),
            pl.BlockSpec((2, _H), lambda i, l: (0, 0)),
            pl.BlockSpec((2 * _D, _H), lambda i, l: (0, 0)),
            pl.BlockSpec((_H, _C), lambda i, l: (0, 0)),
            pl.BlockSpec((1, _C), lambda i, l: (0, 0)),
        ],
        out_specs=pl.BlockSpec((_BB, _C), lambda i, l: (i, 0)),
        out_shape=jax.ShapeDtypeStruct((_B, _C), jnp.float32),
        scratch_shapes=[pltpu.VMEM((_BB, _H), jnp.float32)],
    )(flat, mask, ss, w1t, wft, bfr)


def kernel(token_ids, word_ids, attention_mask, emb1, emb2, W1, b1, gamma,
           beta, Wf, bf):
    del b1
    tok = token_ids.T.reshape(_NW, _NCH, _CH).astype(jnp.int32)
    word = word_ids.T.reshape(_NW, _NCH, _CH).astype(jnp.int32)
    flat = _sc_gather(tok, word, emb1, emb2)
    w1t = W1.T
    ss = _stats(flat, w1t, gamma.reshape(1, _H), beta.reshape(1, _H))
    logits = _main(flat, attention_mask, ss, w1t, Wf.T, bf.reshape(1, _C))
    return logits
